# baseline (speedup 1.0000x reference)
import functools

import jax
import jax.numpy as jnp
from jax import lax
from jax.experimental import pallas as pl
from jax.experimental.pallas import tpu as pltpu
from jax.experimental.pallas import tpu_sc as plsc

_RATE = 0.5
_NUM_CORES = 2
_NUM_SUBCORES = 16
_NUM_WORKERS = _NUM_CORES * _NUM_SUBCORES


def kernel(x):
    B, old_len, D = x.shape
    new_len = int(old_len * _RATE)
    R = B * new_len
    x3 = x.reshape(R, 2, D)

    rows_per_w = R // _NUM_WORKERS

    mesh = plsc.VectorSubcoreMesh(core_axis_name="c", subcore_axis_name="s")

    @functools.partial(
        pl.kernel,
        out_type=jax.ShapeDtypeStruct((R, 1, D), jnp.float32),
        mesh=mesh,
    )
    def run(x_hbm, o_hbm):
        wid = lax.axis_index("s") * _NUM_CORES + lax.axis_index("c")
        base = wid * rows_per_w
        pltpu.sync_copy(
            x_hbm.at[pl.ds(base, rows_per_w), 0:1],
            o_hbm.at[pl.ds(base, rows_per_w)],
        )

    out = run(x3)
    return out.reshape(B, new_len, D)


# speedup vs baseline: 8.9057x; 8.9057x over previous
import functools

import jax
import jax.numpy as jnp
from jax import lax
from jax.experimental import pallas as pl
from jax.experimental.pallas import tpu as pltpu
from jax.experimental.pallas import tpu_sc as plsc

_RATE = 0.5
_NUM_CORES = 2
_NUM_SUBCORES = 16
_NUM_WORKERS = _NUM_CORES * _NUM_SUBCORES
_CHUNK = 16
_NBUF = 3


def kernel(x):
    B, old_len, D = x.shape
    new_len = int(old_len * _RATE)
    R = B * new_len
    x3 = x.reshape(R, 2, D)

    rows_per_w = R // _NUM_WORKERS
    n_iter = rows_per_w // _CHUNK

    mesh = plsc.VectorSubcoreMesh(core_axis_name="c", subcore_axis_name="s")

    @functools.partial(
        pl.kernel,
        out_type=jax.ShapeDtypeStruct((R, 1, D), jnp.float32),
        mesh=mesh,
        scratch_types=(
            [pltpu.VMEM((_CHUNK, 1, D), jnp.float32) for _ in range(_NBUF)]
            + [pltpu.SemaphoreType.DMA for _ in range(2 * _NBUF)]
        ),
    )
    def run(x_hbm, o_hbm, *scratch):
        bufs = scratch[:_NBUF]
        in_sems = scratch[_NBUF : 2 * _NBUF]
        out_sems = scratch[2 * _NBUF :]
        wid = lax.axis_index("s") * _NUM_CORES + lax.axis_index("c")
        base = wid * rows_per_w

        def start_in(i):
            return pltpu.async_copy(
                x_hbm.at[pl.ds(base + i * _CHUNK, _CHUNK), 0:1],
                bufs[i % _NBUF],
                in_sems[i % _NBUF],
            )

        def start_out(i):
            return pltpu.async_copy(
                bufs[i % _NBUF],
                o_hbm.at[pl.ds(base + i * _CHUNK, _CHUNK)],
                out_sems[i % _NBUF],
            )

        h_in = {}
        h_out = {}
        waited = set()
        for j in range(min(_NBUF, n_iter)):
            h_in[j] = start_in(j)
        for i in range(n_iter):
            h_in[i].wait()
            h_out[i] = start_out(i)
            nxt = i + _NBUF
            if nxt < n_iter:
                h_out[i].wait()
                waited.add(i)
                h_in[nxt] = start_in(nxt)
        for i in range(n_iter):
            if i not in waited:
                h_out[i].wait()

    out = run(x3)
    return out.reshape(B, new_len, D)


# speedup vs baseline: 9.0387x; 1.0149x over previous
import functools

import jax
import jax.numpy as jnp
from jax import lax
from jax.experimental import pallas as pl
from jax.experimental.pallas import tpu as pltpu
from jax.experimental.pallas import tpu_sc as plsc

_RATE = 0.5
_NUM_CORES = 2
_NUM_SUBCORES = 16
_NUM_WORKERS = _NUM_CORES * _NUM_SUBCORES
_CHUNK = 16
_NBUF = 3
_LAG = 1


def kernel(x):
    B, old_len, D = x.shape
    new_len = int(old_len * _RATE)
    R = B * new_len
    x3 = x.reshape(R, 2, D)

    rows_per_w = R // _NUM_WORKERS
    n_iter = rows_per_w // _CHUNK

    mesh = plsc.VectorSubcoreMesh(core_axis_name="c", subcore_axis_name="s")

    @functools.partial(
        pl.kernel,
        out_type=jax.ShapeDtypeStruct((R, 1, D), jnp.float32),
        mesh=mesh,
        scratch_types=(
            [pltpu.VMEM_SHARED((_NUM_SUBCORES, _NBUF, _CHUNK, 1, D), jnp.float32)]
            + [pltpu.SemaphoreType.DMA for _ in range(2 * _NBUF)]
        ),
    )
    def run(x_hbm, o_hbm, shared, *sems):
        in_sems = sems[:_NBUF]
        out_sems = sems[_NBUF:]
        cid = lax.axis_index("c")
        sid = lax.axis_index("s")
        wid = sid * _NUM_CORES + cid
        base = wid * rows_per_w

        def start_in(i):
            return pltpu.async_copy(
                x_hbm.at[pl.ds(base + i * _CHUNK, _CHUNK), 0:1],
                shared.at[sid, i % _NBUF],
                in_sems[i % _NBUF],
            )

        def start_out(i):
            return pltpu.async_copy(
                shared.at[sid, i % _NBUF],
                o_hbm.at[pl.ds(base + i * _CHUNK, _CHUNK)],
                out_sems[i % _NBUF],
            )

        h_in = {}
        h_out = {}
        waited = set()
        for j in range(min(_NBUF, n_iter)):
            h_in[j] = start_in(j)
        for i in range(n_iter):
            h_in[i].wait()
            h_out[i] = start_out(i)
            j = i - _LAG
            if j >= 0 and j + _NBUF < n_iter:
                h_out[j].wait()
                waited.add(j)
                h_in[j + _NBUF] = start_in(j + _NBUF)
        for i in range(n_iter):
            if i not in waited:
                h_out[i].wait()

    out = run(x3)
    return out.reshape(B, new_len, D)


# speedup vs baseline: 38.8588x; 4.2991x over previous
import functools

import jax
import jax.numpy as jnp
from jax import lax
from jax.experimental import pallas as pl
from jax.experimental.pallas import tpu as pltpu
from jax.experimental.pallas import tpu_sc as plsc

_RATE = 0.5
_NUM_CORES = 2
_NUM_SUBCORES = 16
_NUM_WORKERS = _NUM_CORES * _NUM_SUBCORES
_CHUNK = 16
_NBUF = 3
_LAG = 1


def kernel(x):
    B, old_len, D = x.shape
    new_len = int(old_len * _RATE)
    R = B * new_len
    x2 = x.reshape(B * old_len, D)
    src_rows = jnp.arange(R, dtype=jnp.int32) * 2

    rows_per_w = R // _NUM_WORKERS
    n_iter = rows_per_w // _CHUNK

    mesh = plsc.VectorSubcoreMesh(core_axis_name="c", subcore_axis_name="s")

    @functools.partial(
        pl.kernel,
        out_type=jax.ShapeDtypeStruct((R, D), jnp.float32),
        mesh=mesh,
        scratch_types=(
            [pltpu.VMEM((rows_per_w,), jnp.int32)]
            + [pltpu.VMEM((_CHUNK, D), jnp.float32) for _ in range(_NBUF)]
            + [pltpu.SemaphoreType.DMA for _ in range(2 * _NBUF)]
        ),
    )
    def run(x_hbm, idx_hbm, o_hbm, idx_v, *scratch):
        bufs = scratch[:_NBUF]
        in_sems = scratch[_NBUF : 2 * _NBUF]
        out_sems = scratch[2 * _NBUF :]
        wid = lax.axis_index("s") * _NUM_CORES + lax.axis_index("c")
        base = wid * rows_per_w
        pltpu.sync_copy(idx_hbm.at[pl.ds(base, rows_per_w)], idx_v)

        def start_in(i):
            return pltpu.async_copy(
                x_hbm.at[idx_v.at[pl.ds(i * _CHUNK, _CHUNK)]],
                bufs[i % _NBUF],
                in_sems[i % _NBUF],
            )

        def start_out(i):
            return pltpu.async_copy(
                bufs[i % _NBUF],
                o_hbm.at[pl.ds(base + i * _CHUNK, _CHUNK)],
                out_sems[i % _NBUF],
            )

        h_in = {}
        h_out = {}
        waited = set()
        for j in range(min(_NBUF, n_iter)):
            h_in[j] = start_in(j)
        for i in range(n_iter):
            h_in[i].wait()
            h_out[i] = start_out(i)
            j = i - _LAG
            if j >= 0 and j + _NBUF < n_iter:
                h_out[j].wait()
                waited.add(j)
                h_in[j + _NBUF] = start_in(j + _NBUF)
        for i in range(n_iter):
            if i not in waited:
                h_out[i].wait()

    out = run(x2, src_rows)
    return out.reshape(B, new_len, D)


# speedup vs baseline: 38.9413x; 1.0021x over previous
import functools

import jax
import jax.numpy as jnp
from jax import lax
from jax.experimental import pallas as pl
from jax.experimental.pallas import tpu as pltpu
from jax.experimental.pallas import tpu_sc as plsc

_RATE = 0.5
_NUM_CORES = 2
_NUM_SUBCORES = 16
_NUM_WORKERS = _NUM_CORES * _NUM_SUBCORES
_CHUNK = 8
_NBUF = 6
_LAG = 3


def kernel(x):
    B, old_len, D = x.shape
    new_len = int(old_len * _RATE)
    R = B * new_len
    x2 = x.reshape(B * old_len, D)
    src_rows = jnp.arange(R, dtype=jnp.int32) * 2

    rows_per_w = R // _NUM_WORKERS
    n_iter = rows_per_w // _CHUNK

    mesh = plsc.VectorSubcoreMesh(core_axis_name="c", subcore_axis_name="s")

    @functools.partial(
        pl.kernel,
        out_type=jax.ShapeDtypeStruct((R, D), jnp.float32),
        mesh=mesh,
        scratch_types=(
            [pltpu.VMEM((rows_per_w,), jnp.int32)]
            + [pltpu.VMEM((_CHUNK, D), jnp.float32) for _ in range(_NBUF)]
            + [pltpu.SemaphoreType.DMA for _ in range(2 * _NBUF)]
        ),
    )
    def run(x_hbm, idx_hbm, o_hbm, idx_v, *scratch):
        bufs = scratch[:_NBUF]
        in_sems = scratch[_NBUF : 2 * _NBUF]
        out_sems = scratch[2 * _NBUF :]
        wid = lax.axis_index("s") * _NUM_CORES + lax.axis_index("c")
        base = wid * rows_per_w
        pltpu.sync_copy(idx_hbm.at[pl.ds(base, rows_per_w)], idx_v)

        def start_in(i):
            return pltpu.async_copy(
                x_hbm.at[idx_v.at[pl.ds(i * _CHUNK, _CHUNK)]],
                bufs[i % _NBUF],
                in_sems[i % _NBUF],
            )

        def start_out(i):
            return pltpu.async_copy(
                bufs[i % _NBUF],
                o_hbm.at[pl.ds(base + i * _CHUNK, _CHUNK)],
                out_sems[i % _NBUF],
            )

        h_in = {}
        h_out = {}
        waited = set()
        for j in range(min(_NBUF, n_iter)):
            h_in[j] = start_in(j)
        for i in range(n_iter):
            h_in[i].wait()
            h_out[i] = start_out(i)
            j = i - _LAG
            if j >= 0 and j + _NBUF < n_iter:
                h_out[j].wait()
                waited.add(j)
                h_in[j + _NBUF] = start_in(j + _NBUF)
        for i in range(n_iter):
            if i not in waited:
                h_out[i].wait()

    out = run(x2, src_rows)
    return out.reshape(B, new_len, D)
